# baseline (speedup 1.0000x reference)
import functools

import jax
import jax.numpy as jnp
from jax import lax
from jax.experimental import pallas as pl
from jax.experimental.pallas import tpu as pltpu
from jax.experimental.pallas import tpu_sc as plsc

EMBED_DIM = 16
PACK = 128 // EMBED_DIM
NUM_SC_CORES = 2
NUM_SC_SUBCORES = 16
NUM_WORKERS = NUM_SC_CORES * NUM_SC_SUBCORES
CHUNK = 256


def _sc_gather(mrow, urow, movie_packed, user_packed):
    batch = mrow.shape[0]
    b_per_w = batch // NUM_WORKERS
    mesh = plsc.VectorSubcoreMesh(core_axis_name="c", subcore_axis_name="s")

    @functools.partial(
        pl.kernel,
        mesh=mesh,
        out_type=(
            jax.ShapeDtypeStruct((batch, 128), jnp.float32),
            jax.ShapeDtypeStruct((batch, 128), jnp.float32),
        ),
        scratch_types=[
            pltpu.VMEM((b_per_w,), jnp.int32),
            pltpu.VMEM((b_per_w,), jnp.int32),
            pltpu.VMEM((CHUNK, 128), jnp.float32),
            pltpu.VMEM((CHUNK, 128), jnp.float32),
            pltpu.SemaphoreType.DMA,
            pltpu.SemaphoreType.DMA,
        ],
    )
    def gather_kernel(mt_hbm, ut_hbm, mi_hbm, ui_hbm, mo_hbm, uo_hbm,
                      mi_v, ui_v, mrows_v, urows_v, sem_m, sem_u):
        wid = lax.axis_index("s") * NUM_SC_CORES + lax.axis_index("c")
        base = wid * b_per_w
        pltpu.sync_copy(mi_hbm.at[pl.ds(base, b_per_w)], mi_v)
        pltpu.sync_copy(ui_hbm.at[pl.ds(base, b_per_w)], ui_v)

        @pl.loop(0, b_per_w, step=CHUNK)
        def _(c):
            cm = pltpu.async_copy(mt_hbm.at[mi_v.at[pl.ds(c, CHUNK)]],
                                  mrows_v, sem_m)
            cu = pltpu.async_copy(ut_hbm.at[ui_v.at[pl.ds(c, CHUNK)]],
                                  urows_v, sem_u)
            cm.wait()
            cu.wait()
            pltpu.sync_copy(mrows_v, mo_hbm.at[pl.ds(base + c, CHUNK)])
            pltpu.sync_copy(urows_v, uo_hbm.at[pl.ds(base + c, CHUNK)])

    return gather_kernel(movie_packed, user_packed, mrow, urow)


def _mlp_body(mc_ref, uc_ref, msub_ref, usub_ref, w0m_ref, w0u_ref, b0_ref,
              w1_ref, b1_ref, wo_ref, bo_ref, o_ref):
    col_group = jax.lax.broadcasted_iota(jnp.int32, (1, 128), 1) // EMBED_DIM
    mm = jnp.where(msub_ref[...] == col_group, mc_ref[...], 0.0)
    uu = jnp.where(usub_ref[...] == col_group, uc_ref[...], 0.0)
    h = (jnp.dot(mm, w0m_ref[...], preferred_element_type=jnp.float32)
         + jnp.dot(uu, w0u_ref[...], preferred_element_type=jnp.float32)
         + b0_ref[...])
    h = jnp.maximum(h, 0.0)
    h = jnp.dot(h, w1_ref[...], preferred_element_type=jnp.float32) + b1_ref[...]
    h = jnp.maximum(h, 0.0)
    o = jnp.dot(h, wo_ref[...], preferred_element_type=jnp.float32) + bo_ref[...]
    o_ref[...] = jax.nn.sigmoid(o)


def kernel(movie_id, user_id, movie_table, user_table, W0, b0, W1, b1, Wo, bo):
    batch = movie_id.shape[0]
    movie_id = movie_id.astype(jnp.int32)
    user_id = user_id.astype(jnp.int32)
    mrow = movie_id // PACK
    urow = user_id // PACK
    msub = (movie_id % PACK)[:, None]
    usub = (user_id % PACK)[:, None]
    movie_packed = movie_table.reshape(-1, 128)
    user_packed = user_table.reshape(-1, 128)
    mc, uc = _sc_gather(mrow, urow, movie_packed, user_packed)
    w0m = jnp.tile(W0[:, :EMBED_DIM].T, (PACK, 1))
    w0u = jnp.tile(W0[:, EMBED_DIM:].T, (PACK, 1))
    out = pl.pallas_call(
        _mlp_body,
        out_shape=jax.ShapeDtypeStruct((batch, 1), jnp.float32),
    )(mc, uc, msub, usub, w0m, w0u, b0[None, :], W1.T, b1[None, :],
      Wo.T, bo[None, :])
    return out


# speedup vs baseline: 1.0263x; 1.0263x over previous
import functools

import jax
import jax.numpy as jnp
from jax import lax
from jax.experimental import pallas as pl
from jax.experimental.pallas import tpu as pltpu
from jax.experimental.pallas import tpu_sc as plsc

EMBED_DIM = 16
NUM_SC_CORES = 2
NUM_SC_SUBCORES = 16
NUM_WORKERS = NUM_SC_CORES * NUM_SC_SUBCORES


def _sc_gather(movie_id, user_id, movie_table, user_table):
    batch = movie_id.shape[0]
    b_per_w = batch // NUM_WORKERS
    mesh = plsc.VectorSubcoreMesh(core_axis_name="c", subcore_axis_name="s")

    @functools.partial(
        pl.kernel,
        mesh=mesh,
        out_type=(
            jax.ShapeDtypeStruct((batch, EMBED_DIM), jnp.float32),
            jax.ShapeDtypeStruct((batch, EMBED_DIM), jnp.float32),
        ),
        scratch_types=[
            pltpu.VMEM((b_per_w,), jnp.int32),
            pltpu.VMEM((b_per_w,), jnp.int32),
            pltpu.VMEM((b_per_w, EMBED_DIM), jnp.float32),
            pltpu.VMEM((b_per_w, EMBED_DIM), jnp.float32),
            pltpu.SemaphoreType.DMA,
            pltpu.SemaphoreType.DMA,
        ],
        compiler_params=pltpu.CompilerParams(use_tc_tiling_on_sc=False),
    )
    def gather_kernel(mt_hbm, ut_hbm, mi_hbm, ui_hbm, mo_hbm, uo_hbm,
                      mi_v, ui_v, mrows_v, urows_v, sem_m, sem_u):
        wid = lax.axis_index("s") * NUM_SC_CORES + lax.axis_index("c")
        base = wid * b_per_w
        pltpu.sync_copy(mi_hbm.at[pl.ds(base, b_per_w)], mi_v)
        pltpu.sync_copy(ui_hbm.at[pl.ds(base, b_per_w)], ui_v)
        cm = pltpu.async_copy(mt_hbm.at[mi_v], mrows_v, sem_m)
        cu = pltpu.async_copy(ut_hbm.at[ui_v], urows_v, sem_u)
        cm.wait()
        cu.wait()
        pltpu.sync_copy(mrows_v, mo_hbm.at[pl.ds(base, b_per_w)])
        pltpu.sync_copy(urows_v, uo_hbm.at[pl.ds(base, b_per_w)])

    return gather_kernel(movie_table, user_table, movie_id, user_id)


def _mlp_body(me_ref, ue_ref, w0m_ref, w0u_ref, b0_ref, w1_ref, b1_ref,
              wo_ref, bo_ref, o_ref):
    h = (jnp.dot(me_ref[...], w0m_ref[...], preferred_element_type=jnp.float32)
         + jnp.dot(ue_ref[...], w0u_ref[...], preferred_element_type=jnp.float32)
         + b0_ref[...])
    h = jnp.maximum(h, 0.0)
    h = jnp.dot(h, w1_ref[...], preferred_element_type=jnp.float32) + b1_ref[...]
    h = jnp.maximum(h, 0.0)
    o = jnp.dot(h, wo_ref[...], preferred_element_type=jnp.float32) + bo_ref[...]
    o_ref[...] = jax.nn.sigmoid(o)


def kernel(movie_id, user_id, movie_table, user_table, W0, b0, W1, b1, Wo, bo):
    batch = movie_id.shape[0]
    movie_id = movie_id.astype(jnp.int32)
    user_id = user_id.astype(jnp.int32)
    me, ue = _sc_gather(movie_id, user_id, movie_table, user_table)
    w0m = W0[:, :EMBED_DIM].T
    w0u = W0[:, EMBED_DIM:].T
    out = pl.pallas_call(
        _mlp_body,
        out_shape=jax.ShapeDtypeStruct((batch, 1), jnp.float32),
    )(me, ue, w0m, w0u, b0[None, :], W1.T, b1[None, :], Wo.T, bo[None, :])
    return out
